# baseline (speedup 1.0000x reference)
import jax
import jax.numpy as jnp
from jax.experimental import pallas as pl

B, DEG, IN_DIM, OUT_DIM, E = 10000, 16, 128, 128, 4
BB = 400


def _fused_body(lab_ref, hw_ref, wcat_ref, out_ref):
    hwb = hw_ref[...]
    lab = lab_ref[...]
    S = jnp.sum(hwb, axis=1)
    Ts = []
    for e in range(E - 1):
        m = lab == e
        Ts.append(jnp.sum(jnp.where(m, hwb, 0.0), axis=1))
    Ts.append(S - Ts[0] - Ts[1] - Ts[2])
    Vs = []
    for e in range(E):
        pe = jnp.max((lab == e).astype(jnp.float32), axis=1)
        Vs.append(pe * S - Ts[e])
    V = jnp.concatenate(Vs, axis=1)
    out_ref[...] = jnp.dot(V, wcat_ref[...],
                           preferred_element_type=jnp.float32)


def kernel(hw, elabels, W):
    wcat = W.transpose(0, 2, 1).reshape(E * IN_DIM, OUT_DIM)
    out = pl.pallas_call(
        _fused_body,
        grid=(B // BB,),
        in_specs=[
            pl.BlockSpec((BB, DEG, 1), lambda i: (i, 0, 0)),
            pl.BlockSpec((BB, DEG, IN_DIM), lambda i: (i, 0, 0)),
            pl.BlockSpec((E * IN_DIM, OUT_DIM), lambda i: (0, 0)),
        ],
        out_specs=pl.BlockSpec((BB, OUT_DIM), lambda i: (i, 0)),
        out_shape=jax.ShapeDtypeStruct((B, OUT_DIM), jnp.float32),
    )(elabels.reshape(B, DEG, 1), hw, wcat)
    return out.reshape(-1)


# speedup vs baseline: 1.4695x; 1.4695x over previous
import jax
import jax.numpy as jnp
from jax.experimental import pallas as pl

B, DEG, IN_DIM, OUT_DIM, E = 10000, 16, 128, 128, 4
BB = 80


def _fused_body(lab_ref, hw_ref, wcat_ref, out_ref):
    lab = lab_ref[...]
    powv = jnp.left_shift(jnp.ones_like(lab), lab * 5)
    bits = jnp.sum(powv, axis=1, keepdims=True)

    S = None
    T = [None, None, None]
    for j in range(DEG):
        rowj = hw_ref[:, j, :]
        labj = jnp.broadcast_to(lab[:, j:j + 1], (BB, IN_DIM))
        S = rowj if S is None else S + rowj
        for e in range(E - 1):
            sel = jnp.where(labj == e, rowj, 0.0)
            T[e] = sel if T[e] is None else T[e] + sel
    T.append(S - T[0] - T[1] - T[2])

    Vs = []
    for e in range(E):
        cnt = jnp.bitwise_and(jnp.right_shift(bits, 5 * e), 31)
        pf = (cnt > 0).astype(jnp.float32)
        Vs.append(pf * S - T[e])
    V = jnp.concatenate(Vs, axis=1)
    out_ref[...] = jnp.dot(V, wcat_ref[...],
                           preferred_element_type=jnp.float32)


def kernel(hw, elabels, W):
    wcat = W.transpose(0, 2, 1).reshape(E * IN_DIM, OUT_DIM)
    out = pl.pallas_call(
        _fused_body,
        grid=(B // BB,),
        in_specs=[
            pl.BlockSpec((BB, DEG), lambda i: (i, 0)),
            pl.BlockSpec((BB, DEG, IN_DIM), lambda i: (i, 0, 0)),
            pl.BlockSpec((E * IN_DIM, OUT_DIM), lambda i: (0, 0)),
        ],
        out_specs=pl.BlockSpec((BB, OUT_DIM), lambda i: (i, 0)),
        out_shape=jax.ShapeDtypeStruct((B, OUT_DIM), jnp.float32),
    )(elabels, hw, wcat)
    return out.reshape(-1)
